# baseline (speedup 1.0000x reference)
import functools

import jax
import jax.numpy as jnp
from jax import lax
from jax.experimental import pallas as pl
from jax.experimental.pallas import tpu as pltpu
from jax.experimental.pallas import tpu_sc as plsc

B, T, C, H = 128, 3, 83, 128
N = B * T * C
E = 16 * N
NCH = 4
CW = H // NCH
NT = 16
ROWS_PT = N // NT
EG = E // 128
GB = 6
NB_ALL = EG // GB
NBT = 41
NEXTRA = NB_ALL - NBT * NT
NPAIR = (NBT - 1) // 2
NBLK = 16
RB = N // NBLK
GGB = 16
GRB = GGB * C


def _sc_segment_sum(src2, dst2, hch, zch):
    mesh = plsc.VectorSubcoreMesh(core_axis_name="c", subcore_axis_name="s")

    @functools.partial(
        pl.kernel,
        out_type=jax.ShapeDtypeStruct((NCH * N, CW), jnp.float32),
        mesh=mesh,
        compiler_params=pltpu.CompilerParams(use_tc_tiling_on_sc=False),
        scratch_types=[
            pltpu.VMEM_SHARED((N, CW), jnp.float32),
            pltpu.VMEM((2, GB, 128), jnp.int32),
            pltpu.VMEM((2, GB, 128), jnp.int32),
            pltpu.VMEM((2, GB, 128, CW), jnp.float32),
            pltpu.SemaphoreType.DMA,
            pltpu.SemaphoreType.DMA,
            pltpu.SemaphoreType.DMA,
        ],
    )
    def k(src_h, dst_h, h_h, z_h, agg_h, accum, sv, dv, rows, isem, gsem, ssem):
        sc = lax.axis_index("c")
        tid = lax.axis_index("s")
        r0 = tid * ROWS_PT

        for j in range(2):
            c = sc * 2 + j

            def idx_fire(bi, p):
                bg = jnp.minimum(tid + bi * NT, NB_ALL - 1)
                g0 = bg * GB
                pltpu.async_copy(src_h.at[pl.ds(c * EG + g0, GB)], sv.at[p], isem)
                pltpu.async_copy(dst_h.at[pl.ds(g0, GB)], dv.at[p], isem)

            def idx_drain(p):
                pltpu.make_async_copy(src_h.at[pl.ds(0, GB)], sv.at[p], isem).wait()
                pltpu.make_async_copy(dst_h.at[pl.ds(0, GB)], dv.at[p], isem).wait()

            def gat(p):
                descs = [pltpu.async_copy(h_h.at[sv.at[p, r]], rows.at[p, r], gsem)
                         for r in range(GB)]
                for d in descs:
                    d.wait()

            def sct_fire(p):
                for r in range(GB):
                    pltpu.async_copy(rows.at[p, r], accum.at[dv.at[p, r]],
                                     ssem, add=True)

            def sct_drain(p):
                for r in range(GB):
                    pltpu.make_async_copy(z_h.at[pl.ds(0, 128)],
                                          rows.at[p, r], ssem).wait()

            pltpu.sync_copy(z_h.at[pl.ds(r0, ROWS_PT)], accum.at[pl.ds(r0, ROWS_PT)])
            plsc.subcore_barrier()

            idx_fire(0, 0)
            idx_drain(0)
            gat(0)
            sct_fire(0)
            idx_fire(1, 1)

            def pair(kk, carry):
                idx_drain(1)
                ga = [pltpu.async_copy(h_h.at[sv.at[1, r]], rows.at[1, r], gsem)
                      for r in range(GB)]
                sct_drain(0)
                idx_fire(2 * kk + 2, 0)
                for d in ga:
                    d.wait()
                sct_fire(1)
                idx_drain(0)
                gb = [pltpu.async_copy(h_h.at[sv.at[0, r]], rows.at[0, r], gsem)
                      for r in range(GB)]
                sct_drain(1)
                idx_fire(2 * kk + 3, 1)
                for d in gb:
                    d.wait()
                sct_fire(0)
                return carry

            lax.fori_loop(0, NPAIR, pair, 0)

            sct_drain(0)
            idx_drain(1)

            @pl.when(tid < NEXTRA)
            def _tail():
                gat(1)
                sct_fire(1)
                sct_drain(1)

            plsc.subcore_barrier()
            pltpu.sync_copy(accum.at[pl.ds(r0, ROWS_PT)],
                            agg_h.at[pl.ds(c * N + r0, ROWS_PT)])
            plsc.subcore_barrier()

    return k(src2, dst2, hch, zch)


def _stats_pair(t):
    s1 = jnp.sum(t, axis=0, keepdims=True)
    s2 = jnp.sum(t * t, axis=0, keepdims=True)
    return jnp.concatenate([s1[None], s2[None]], axis=1)


def _bn_apply(t, st_all, g, bb):
    mu = jnp.sum(st_all[:, 0, :], axis=0, keepdims=True) / N
    var = jnp.sum(st_all[:, 1, :], axis=0, keepdims=True) / N - mu * mu
    inv = lax.rsqrt(var + 1e-5)
    return (t - mu) * inv * g + bb


def _tc_a(h, agg4, w1, b1, eps):
    def body(h_ref, a_ref, w_ref, b_ref, e_ref, t_ref, st_ref):
        agg = jnp.concatenate([a_ref[c] for c in range(NCH)], axis=1)
        z = (1.0 + e_ref[0, 0]) * h_ref[...] + agg
        t = jnp.dot(z, w_ref[...], preferred_element_type=jnp.float32) + b_ref[...]
        t_ref[...] = t
        st_ref[...] = _stats_pair(t)

    return pl.pallas_call(
        body,
        grid=(NBLK,),
        in_specs=[
            pl.BlockSpec((RB, H), lambda i: (i, 0)),
            pl.BlockSpec((NCH, RB, CW), lambda i: (0, i, 0)),
            pl.BlockSpec((H, H), lambda i: (0, 0)),
            pl.BlockSpec((1, H), lambda i: (0, 0)),
            pl.BlockSpec(memory_space=pltpu.SMEM),
        ],
        out_specs=[
            pl.BlockSpec((RB, H), lambda i: (i, 0)),
            pl.BlockSpec((1, 2, H), lambda i: (i, 0, 0)),
        ],
        out_shape=[
            jax.ShapeDtypeStruct((N, H), jnp.float32),
            jax.ShapeDtypeStruct((NBLK, 2, H), jnp.float32),
        ],
    )(h, agg4, w1, b1, eps)


def _tc_b(t, st, g1, bb1, w2, b2):
    def body(t_ref, st_ref, g_ref, bb_ref, w_ref, b_ref, s_ref, st2_ref):
        u = jnp.maximum(_bn_apply(t_ref[...], st_ref[...], g_ref[...], bb_ref[...]), 0.0)
        s = jnp.dot(u, w_ref[...], preferred_element_type=jnp.float32) + b_ref[...]
        s_ref[...] = s
        st2_ref[...] = _stats_pair(s)

    return pl.pallas_call(
        body,
        grid=(NBLK,),
        in_specs=[
            pl.BlockSpec((RB, H), lambda i: (i, 0)),
            pl.BlockSpec((NBLK, 2, H), lambda i: (0, 0, 0)),
            pl.BlockSpec((1, H), lambda i: (0, 0)),
            pl.BlockSpec((1, H), lambda i: (0, 0)),
            pl.BlockSpec((H, H), lambda i: (0, 0)),
            pl.BlockSpec((1, H), lambda i: (0, 0)),
        ],
        out_specs=[
            pl.BlockSpec((RB, H), lambda i: (i, 0)),
            pl.BlockSpec((1, 2, H), lambda i: (i, 0, 0)),
        ],
        out_shape=[
            jax.ShapeDtypeStruct((N, H), jnp.float32),
            jax.ShapeDtypeStruct((NBLK, 2, H), jnp.float32),
        ],
    )(t, st, g1, bb1, w2, b2)


def _tc_c(s, st2, g, bb):
    def body(s_ref, st_ref, g_ref, bb_ref, h_ref):
        hv = jnp.maximum(_bn_apply(s_ref[...], st_ref[...], g_ref[...], bb_ref[...]), 0.0)
        h_ref[...] = hv

    return pl.pallas_call(
        body,
        grid=(NBLK,),
        in_specs=[
            pl.BlockSpec((RB, H), lambda i: (i, 0)),
            pl.BlockSpec((NBLK, 2, H), lambda i: (0, 0, 0)),
            pl.BlockSpec((1, H), lambda i: (0, 0)),
            pl.BlockSpec((1, H), lambda i: (0, 0)),
        ],
        out_specs=pl.BlockSpec((RB, H), lambda i: (i, 0)),
        out_shape=jax.ShapeDtypeStruct((N, H), jnp.float32),
    )(s, st2, g, bb)


def _tc_gru_pool(xseq, wiht, whht, b_ih, b_hh, wa1, ba1, wa2, ba2, we, be):
    def body(x_ref, wi_ref, wh_ref, bi_ref, bh_ref, a1_ref, ba1_ref,
             a2_ref, ba2_ref, we_ref, be_ref, o_ref):
        hs = jnp.zeros((GRB, H), jnp.float32)
        for t in range(T):
            xt = x_ref[t]
            gi = jnp.dot(xt, wi_ref[...], preferred_element_type=jnp.float32) + bi_ref[...]
            gh = jnp.dot(hs, wh_ref[...], preferred_element_type=jnp.float32) + bh_ref[...]
            r = jax.nn.sigmoid(gi[:, :H] + gh[:, :H])
            zt = jax.nn.sigmoid(gi[:, H:2 * H] + gh[:, H:2 * H])
            n = jnp.tanh(gi[:, 2 * H:] + r * gh[:, 2 * H:])
            hs = (1.0 - zt) * n + zt * hs
        a = jnp.dot(jnp.tanh(
            jnp.dot(hs, a1_ref[...], preferred_element_type=jnp.float32) + ba1_ref[...]),
            a2_ref[...], preferred_element_type=jnp.float32) + ba2_ref[...]
        m = jnp.max(a)
        ex = jnp.exp(a - m)
        rows = lax.broadcasted_iota(jnp.int32, (GRB, GGB), 0) // C
        cols = lax.broadcasted_iota(jnp.int32, (GRB, GGB), 1)
        ind = (rows == cols).astype(jnp.float32)
        denom_g = jnp.dot(ind.T, ex, preferred_element_type=jnp.float32)
        denom = jnp.dot(ind, denom_g, preferred_element_type=jnp.float32)
        w = ex / denom
        pooled = jnp.dot(ind.T, w * hs, preferred_element_type=jnp.float32)
        o_ref[...] = jnp.dot(pooled, we_ref[...], preferred_element_type=jnp.float32) + be_ref[...]

    nblk = (B * C) // GRB
    full = lambda i: (0, 0)
    return pl.pallas_call(
        body,
        grid=(nblk,),
        in_specs=[
            pl.BlockSpec((T, GRB, H), lambda i: (0, i, 0)),
            pl.BlockSpec((H, 3 * H), full),
            pl.BlockSpec((H, 3 * H), full),
            pl.BlockSpec((1, 3 * H), full),
            pl.BlockSpec((1, 3 * H), full),
            pl.BlockSpec((H, H // 2), full),
            pl.BlockSpec((1, H // 2), full),
            pl.BlockSpec((H // 2, 1), full),
            pl.BlockSpec((1, 1), full),
            pl.BlockSpec((H, 64), full),
            pl.BlockSpec((1, 64), full),
        ],
        out_specs=pl.BlockSpec((GGB, 64), lambda i: (i, 0)),
        out_shape=jax.ShapeDtypeStruct((B, 64), jnp.float32),
    )(xseq, wiht, whht, b_ih, b_hh, wa1, ba1, wa2, ba2, we, be)


def kernel(x, params, edge_index, batch):
    src = edge_index[0].astype(jnp.int32)
    dst = edge_index[1].astype(jnp.int32)
    src2 = (src[None, :] + (jnp.arange(NCH, dtype=jnp.int32) * N)[:, None]
            ).reshape(NCH * EG, 128)
    dst2 = dst.reshape(EG, 128)
    zch = jnp.zeros((N, CW), jnp.float32)

    h = x
    hch = x.reshape(N, NCH, CW).transpose(1, 0, 2).reshape(NCH * N, CW)
    for i in range(3):
        aggf = _sc_segment_sum(src2, dst2, hch, zch)
        agg4 = aggf.reshape(NCH, N, CW)
        eps = params[f"eps_{i}"].reshape(1, 1)
        t, st = _tc_a(h, agg4, params[f"W1_{i}"],
                      params[f"b1_{i}"].reshape(1, H), eps)
        s, st2 = _tc_b(t, st, params[f"g1_{i}"].reshape(1, H),
                       params[f"bb1_{i}"].reshape(1, H),
                       params[f"W2_{i}"], params[f"b2_{i}"].reshape(1, H))
        h = _tc_c(s, st2, params[f"g_{i}"].reshape(1, H),
                  params[f"bb_{i}"].reshape(1, H))
        hch = h.reshape(N, NCH, CW).transpose(1, 0, 2).reshape(NCH * N, CW)

    xseq = h.reshape(B, T, C, H).transpose(1, 0, 2, 3).reshape(T, B * C, H)
    return _tc_gru_pool(
        xseq, params["W_ih"].T, params["W_hh"].T,
        params["b_ih"].reshape(1, 3 * H), params["b_hh"].reshape(1, 3 * H),
        params["Wa1"], params["ba1"].reshape(1, H // 2),
        params["Wa2"], params["ba2"].reshape(1, 1),
        params["We"], params["be"].reshape(1, 64))


# speedup vs baseline: 1.1290x; 1.1290x over previous
import functools

import jax
import jax.numpy as jnp
from jax import lax
from jax.experimental import pallas as pl
from jax.experimental.pallas import tpu as pltpu
from jax.experimental.pallas import tpu_sc as plsc

B, T, C, H = 128, 3, 83, 128
N = B * T * C
E = 16 * N
NCH = 4
CW = H // NCH
NT = 16
ROWS_PT = N // NT
EG = E // 128
GB = 6
NB_ALL = EG // GB
NBT = 41
NEXTRA = NB_ALL - NBT * NT
NPAIR = (NBT - 1) // 2
NBLK = 16
RB = N // NBLK
GGB = 16
GRB = GGB * C


def _sc_segment_sum(src2, dst2, hch, zch):
    mesh = plsc.VectorSubcoreMesh(core_axis_name="c", subcore_axis_name="s")

    @functools.partial(
        pl.kernel,
        out_type=jax.ShapeDtypeStruct((N, H), jnp.float32),
        mesh=mesh,
        compiler_params=pltpu.CompilerParams(use_tc_tiling_on_sc=False),
        scratch_types=[
            pltpu.VMEM_SHARED((N, CW), jnp.float32),
            pltpu.VMEM((2, GB, 128), jnp.int32),
            pltpu.VMEM((2, GB, 128), jnp.int32),
            pltpu.VMEM((2, GB, 128, CW), jnp.float32),
            pltpu.SemaphoreType.DMA,
            pltpu.SemaphoreType.DMA,
            pltpu.SemaphoreType.DMA,
        ],
    )
    def k(src_h, dst_h, h_h, z_h, agg_h, accum, sv, dv, rows, isem, gsem, ssem):
        sc = lax.axis_index("c")
        tid = lax.axis_index("s")
        r0 = tid * ROWS_PT

        for j in range(2):
            c = sc * 2 + j

            def idx_fire(bi, p):
                bg = jnp.minimum(tid + bi * NT, NB_ALL - 1)
                g0 = bg * GB
                pltpu.async_copy(src_h.at[pl.ds(c * EG + g0, GB)], sv.at[p], isem)
                pltpu.async_copy(dst_h.at[pl.ds(g0, GB)], dv.at[p], isem)

            def idx_drain(p):
                pltpu.make_async_copy(src_h.at[pl.ds(0, GB)], sv.at[p], isem).wait()
                pltpu.make_async_copy(dst_h.at[pl.ds(0, GB)], dv.at[p], isem).wait()

            def gat(p):
                descs = [pltpu.async_copy(h_h.at[sv.at[p, r]], rows.at[p, r], gsem)
                         for r in range(GB)]
                for d in descs:
                    d.wait()

            def sct_fire(p):
                for r in range(GB):
                    pltpu.async_copy(rows.at[p, r], accum.at[dv.at[p, r]],
                                     ssem, add=True)

            def sct_drain(p):
                for r in range(GB):
                    pltpu.make_async_copy(z_h.at[pl.ds(0, 128)],
                                          rows.at[p, r], ssem).wait()

            pltpu.sync_copy(z_h.at[pl.ds(r0, ROWS_PT)], accum.at[pl.ds(r0, ROWS_PT)])
            plsc.subcore_barrier()

            idx_fire(0, 0)
            idx_drain(0)
            gat(0)
            sct_fire(0)
            idx_fire(1, 1)

            def pair(kk, carry):
                idx_drain(1)
                ga = [pltpu.async_copy(h_h.at[sv.at[1, r]], rows.at[1, r], gsem)
                      for r in range(GB)]
                sct_drain(0)
                idx_fire(2 * kk + 2, 0)
                for d in ga:
                    d.wait()
                sct_fire(1)
                idx_drain(0)
                gb = [pltpu.async_copy(h_h.at[sv.at[0, r]], rows.at[0, r], gsem)
                      for r in range(GB)]
                sct_drain(1)
                idx_fire(2 * kk + 3, 1)
                for d in gb:
                    d.wait()
                sct_fire(0)
                return carry

            lax.fori_loop(0, NPAIR, pair, 0)

            sct_drain(0)
            idx_drain(1)

            @pl.when(tid < NEXTRA)
            def _tail():
                gat(1)
                sct_fire(1)
                sct_drain(1)

            plsc.subcore_barrier()
            pltpu.sync_copy(accum.at[pl.ds(r0, ROWS_PT)],
                            agg_h.at[pl.ds(r0, ROWS_PT), pl.ds(c * CW, CW)])
            plsc.subcore_barrier()

    return k(src2, dst2, hch, zch)


def _stats_pair(t):
    s1 = jnp.sum(t, axis=0, keepdims=True)
    s2 = jnp.sum(t * t, axis=0, keepdims=True)
    return jnp.concatenate([s1[None], s2[None]], axis=1)


def _bn_apply(t, st_all, g, bb):
    mu = jnp.sum(st_all[:, 0, :], axis=0, keepdims=True) / N
    var = jnp.sum(st_all[:, 1, :], axis=0, keepdims=True) / N - mu * mu
    inv = lax.rsqrt(var + 1e-5)
    return (t - mu) * inv * g + bb


def _tc_a(h, agg, w1, b1, eps):
    def body(h_ref, a_ref, w_ref, b_ref, e_ref, t_ref, st_ref):
        z = (1.0 + e_ref[0, 0]) * h_ref[...] + a_ref[...]
        t = jnp.dot(z, w_ref[...], preferred_element_type=jnp.float32) + b_ref[...]
        t_ref[...] = t
        st_ref[...] = _stats_pair(t)

    return pl.pallas_call(
        body,
        grid=(NBLK,),
        in_specs=[
            pl.BlockSpec((RB, H), lambda i: (i, 0)),
            pl.BlockSpec((RB, H), lambda i: (i, 0)),
            pl.BlockSpec((H, H), lambda i: (0, 0)),
            pl.BlockSpec((1, H), lambda i: (0, 0)),
            pl.BlockSpec(memory_space=pltpu.SMEM),
        ],
        out_specs=[
            pl.BlockSpec((RB, H), lambda i: (i, 0)),
            pl.BlockSpec((1, 2, H), lambda i: (i, 0, 0)),
        ],
        out_shape=[
            jax.ShapeDtypeStruct((N, H), jnp.float32),
            jax.ShapeDtypeStruct((NBLK, 2, H), jnp.float32),
        ],
    )(h, agg, w1, b1, eps)


def _tc_b(t, st, g1, bb1, w2, b2):
    def body(t_ref, st_ref, g_ref, bb_ref, w_ref, b_ref, s_ref, st2_ref):
        u = jnp.maximum(_bn_apply(t_ref[...], st_ref[...], g_ref[...], bb_ref[...]), 0.0)
        s = jnp.dot(u, w_ref[...], preferred_element_type=jnp.float32) + b_ref[...]
        s_ref[...] = s
        st2_ref[...] = _stats_pair(s)

    return pl.pallas_call(
        body,
        grid=(NBLK,),
        in_specs=[
            pl.BlockSpec((RB, H), lambda i: (i, 0)),
            pl.BlockSpec((NBLK, 2, H), lambda i: (0, 0, 0)),
            pl.BlockSpec((1, H), lambda i: (0, 0)),
            pl.BlockSpec((1, H), lambda i: (0, 0)),
            pl.BlockSpec((H, H), lambda i: (0, 0)),
            pl.BlockSpec((1, H), lambda i: (0, 0)),
        ],
        out_specs=[
            pl.BlockSpec((RB, H), lambda i: (i, 0)),
            pl.BlockSpec((1, 2, H), lambda i: (i, 0, 0)),
        ],
        out_shape=[
            jax.ShapeDtypeStruct((N, H), jnp.float32),
            jax.ShapeDtypeStruct((NBLK, 2, H), jnp.float32),
        ],
    )(t, st, g1, bb1, w2, b2)


def _tc_c(s, st2, g, bb):
    def body(s_ref, st_ref, g_ref, bb_ref, h_ref, ch_ref):
        hv = jnp.maximum(_bn_apply(s_ref[...], st_ref[...], g_ref[...], bb_ref[...]), 0.0)
        h_ref[...] = hv
        ch_ref[...] = jnp.concatenate(
            [hv[:, c * CW:(c + 1) * CW][None] for c in range(NCH)], axis=0)

    return pl.pallas_call(
        body,
        grid=(NBLK,),
        in_specs=[
            pl.BlockSpec((RB, H), lambda i: (i, 0)),
            pl.BlockSpec((NBLK, 2, H), lambda i: (0, 0, 0)),
            pl.BlockSpec((1, H), lambda i: (0, 0)),
            pl.BlockSpec((1, H), lambda i: (0, 0)),
        ],
        out_specs=[
            pl.BlockSpec((RB, H), lambda i: (i, 0)),
            pl.BlockSpec((NCH, RB, CW), lambda i: (0, i, 0)),
        ],
        out_shape=[
            jax.ShapeDtypeStruct((N, H), jnp.float32),
            jax.ShapeDtypeStruct((NCH, N, CW), jnp.float32),
        ],
    )(s, st2, g, bb)


def _tc_gru_pool(xseq, wiht, whht, b_ih, b_hh, wa1, ba1, wa2, ba2, we, be):
    def body(x_ref, wi_ref, wh_ref, bi_ref, bh_ref, a1_ref, ba1_ref,
             a2_ref, ba2_ref, we_ref, be_ref, o_ref):
        hs = jnp.zeros((GRB, H), jnp.float32)
        for t in range(T):
            xt = x_ref[t]
            gi = jnp.dot(xt, wi_ref[...], preferred_element_type=jnp.float32) + bi_ref[...]
            gh = jnp.dot(hs, wh_ref[...], preferred_element_type=jnp.float32) + bh_ref[...]
            r = jax.nn.sigmoid(gi[:, :H] + gh[:, :H])
            zt = jax.nn.sigmoid(gi[:, H:2 * H] + gh[:, H:2 * H])
            n = jnp.tanh(gi[:, 2 * H:] + r * gh[:, 2 * H:])
            hs = (1.0 - zt) * n + zt * hs
        a = jnp.dot(jnp.tanh(
            jnp.dot(hs, a1_ref[...], preferred_element_type=jnp.float32) + ba1_ref[...]),
            a2_ref[...], preferred_element_type=jnp.float32) + ba2_ref[...]
        m = jnp.max(a)
        ex = jnp.exp(a - m)
        rows = lax.broadcasted_iota(jnp.int32, (GRB, GGB), 0) // C
        cols = lax.broadcasted_iota(jnp.int32, (GRB, GGB), 1)
        ind = (rows == cols).astype(jnp.float32)
        denom_g = jnp.dot(ind.T, ex, preferred_element_type=jnp.float32)
        denom = jnp.dot(ind, denom_g, preferred_element_type=jnp.float32)
        w = ex / denom
        pooled = jnp.dot(ind.T, w * hs, preferred_element_type=jnp.float32)
        o_ref[...] = jnp.dot(pooled, we_ref[...], preferred_element_type=jnp.float32) + be_ref[...]

    nblk = (B * C) // GRB
    full = lambda i: (0, 0)
    return pl.pallas_call(
        body,
        grid=(nblk,),
        in_specs=[
            pl.BlockSpec((T, GRB, H), lambda i: (0, i, 0)),
            pl.BlockSpec((H, 3 * H), full),
            pl.BlockSpec((H, 3 * H), full),
            pl.BlockSpec((1, 3 * H), full),
            pl.BlockSpec((1, 3 * H), full),
            pl.BlockSpec((H, H // 2), full),
            pl.BlockSpec((1, H // 2), full),
            pl.BlockSpec((H // 2, 1), full),
            pl.BlockSpec((1, 1), full),
            pl.BlockSpec((H, 64), full),
            pl.BlockSpec((1, 64), full),
        ],
        out_specs=pl.BlockSpec((GGB, 64), lambda i: (i, 0)),
        out_shape=jax.ShapeDtypeStruct((B, 64), jnp.float32),
    )(xseq, wiht, whht, b_ih, b_hh, wa1, ba1, wa2, ba2, we, be)


def kernel(x, params, edge_index, batch):
    src = edge_index[0].astype(jnp.int32)
    dst = edge_index[1].astype(jnp.int32)
    src2 = (src[None, :] + (jnp.arange(NCH, dtype=jnp.int32) * N)[:, None]
            ).reshape(NCH * EG, 128)
    dst2 = dst.reshape(EG, 128)
    zch = jnp.zeros((N, CW), jnp.float32)

    h = x
    hch = x.reshape(N, NCH, CW).transpose(1, 0, 2).reshape(NCH * N, CW)
    for i in range(3):
        agg = _sc_segment_sum(src2, dst2, hch, zch)
        eps = params[f"eps_{i}"].reshape(1, 1)
        t, st = _tc_a(h, agg, params[f"W1_{i}"],
                      params[f"b1_{i}"].reshape(1, H), eps)
        s, st2 = _tc_b(t, st, params[f"g1_{i}"].reshape(1, H),
                       params[f"bb1_{i}"].reshape(1, H),
                       params[f"W2_{i}"], params[f"b2_{i}"].reshape(1, H))
        h, hch3 = _tc_c(s, st2, params[f"g_{i}"].reshape(1, H),
                        params[f"bb_{i}"].reshape(1, H))
        hch = hch3.reshape(NCH * N, CW)

    xseq = h.reshape(B, T, C, H).transpose(1, 0, 2, 3).reshape(T, B * C, H)
    return _tc_gru_pool(
        xseq, params["W_ih"].T, params["W_hh"].T,
        params["b_ih"].reshape(1, 3 * H), params["b_hh"].reshape(1, 3 * H),
        params["Wa1"], params["ba1"].reshape(1, H // 2),
        params["Wa2"], params["ba2"].reshape(1, 1),
        params["We"], params["be"].reshape(1, 64))
